# baseline (speedup 1.0000x reference)
import functools

import jax
import jax.numpy as jnp
from jax import lax
from jax.experimental import pallas as pl
from jax.experimental.pallas import tpu as pltpu
from jax.experimental.pallas import tpu_sc as plsc

N = 10000
E = 320000
NP = 10112
NC, NS = 2, 16
NW = NC * NS
CH = 128
NB = 5
NCHUNK = 80
NR = NCHUNK // NB
EW = NCHUNK * CH
EPAD = NW * EW
RPT = NP // NS


def _sc_pass(D, with_gather):
    mesh = plsc.VectorSubcoreMesh(core_axis_name="c", subcore_axis_name="s")

    def body(*refs):
        if with_gather:
            (table_hbm, src_hbm, dst_hbm, out_hbm,
             rows_v, srcs_v, dsts_v, acc_sh, gsem, ssem) = refs
        else:
            dst_hbm, out_hbm, rows_v, dsts_v, acc_sh, ssem = refs
        c = lax.axis_index("c")
        s = lax.axis_index("s")
        wid = c * NS + s

        @pl.loop(0, CH)
        def _(i):
            @pl.loop(0, D // 16)
            def _(j):
                rows_v[0, i, pl.ds(j * 16, 16)] = jnp.zeros((16,), jnp.float32)

        r0 = s * RPT

        @pl.loop(0, RPT // CH)
        def _(k):
            pltpu.sync_copy(rows_v.at[0], acc_sh.at[pl.ds(r0 + k * CH, CH)])

        rem = RPT % CH
        if rem:
            pltpu.sync_copy(
                rows_v.at[0, pl.ds(0, rem)],
                acc_sh.at[pl.ds(r0 + (RPT // CH) * CH, rem)],
            )

        if not with_gather:
            @pl.loop(0, CH)
            def _(i):
                @pl.loop(0, D // 16)
                def _(j):
                    rows_v[0, i, pl.ds(j * 16, 16)] = jnp.ones((16,), jnp.float32)

        pltpu.sync_copy(dst_hbm.at[wid], dsts_v)
        if with_gather:
            pltpu.sync_copy(src_hbm.at[wid], srcs_v)

        plsc.subcore_barrier()

        if with_gather:
            for b in range(NB):
                pltpu.async_copy(
                    table_hbm.at[srcs_v.at[b]], rows_v.at[b], gsem.at[b]
                )

            @pl.loop(0, NR)
            def _(r):
                j0 = r * NB
                for b in range(NB):
                    pltpu.make_async_copy(
                        table_hbm.at[srcs_v.at[j0 + b]], rows_v.at[b],
                        gsem.at[b],
                    ).wait()
                    pltpu.async_copy(
                        rows_v.at[b], acc_sh.at[dsts_v.at[j0 + b]],
                        ssem.at[b], add=True,
                    )
                for b in range(NB):
                    pltpu.make_async_copy(
                        rows_v.at[b], acc_sh.at[dsts_v.at[j0 + b]],
                        ssem.at[b],
                    ).wait()

                    @pl.when(r < NR - 1)
                    def _():
                        pltpu.async_copy(
                            table_hbm.at[srcs_v.at[j0 + NB + b]],
                            rows_v.at[b], gsem.at[b],
                        )
        else:
            @pl.loop(0, NR)
            def _(r):
                j0 = r * NB
                for b in range(NB):
                    pltpu.async_copy(
                        rows_v.at[0], acc_sh.at[dsts_v.at[j0 + b]],
                        ssem.at[b], add=True,
                    )
                for b in range(NB):
                    pltpu.make_async_copy(
                        rows_v.at[0], acc_sh.at[dsts_v.at[j0 + b]],
                        ssem.at[b],
                    ).wait()

        plsc.subcore_barrier()

        @pl.loop(0, RPT // CH)
        def _(k):
            pltpu.sync_copy(
                acc_sh.at[pl.ds(r0 + k * CH, CH)],
                out_hbm.at[c, pl.ds(r0 + k * CH, CH)],
            )

        if rem:
            pltpu.sync_copy(
                acc_sh.at[pl.ds(r0 + (RPT // CH) * CH, rem)],
                out_hbm.at[c, pl.ds(r0 + (RPT // CH) * CH, rem)],
            )

    if with_gather:
        scratch = [
            pltpu.VMEM((NB, CH, D), jnp.float32),
            pltpu.VMEM((NCHUNK, CH), jnp.int32),
            pltpu.VMEM((NCHUNK, CH), jnp.int32),
            pltpu.VMEM_SHARED((NP, D), jnp.float32),
            pltpu.SemaphoreType.DMA((NB,)),
            pltpu.SemaphoreType.DMA((NB,)),
        ]
    else:
        scratch = [
            pltpu.VMEM((1, CH, D), jnp.float32),
            pltpu.VMEM((NCHUNK, CH), jnp.int32),
            pltpu.VMEM_SHARED((NP, D), jnp.float32),
            pltpu.SemaphoreType.DMA((NB,)),
        ]
    return pl.kernel(
        body,
        out_type=jax.ShapeDtypeStruct((NC, NP, D), jnp.float32),
        mesh=mesh,
        compiler_params=pltpu.CompilerParams(use_tc_tiling_on_sc=False),
        scratch_types=scratch,
    )


_DOT = functools.partial(
    lax.dot_general,
    dimension_numbers=(((1,), (0,)), ((), ())),
    precision=lax.Precision.HIGHEST,
    preferred_element_type=jnp.float32,
)


BM = 1264
_GRID = NP // BM


def _dis(deg_ref):
    deg = deg_ref[0, :, 0:1] + deg_ref[1, :, 0:1] + 1.0
    r0 = pl.program_id(0) * BM
    rows = r0 + lax.broadcasted_iota(jnp.int32, (BM, 1), 0)
    return jnp.where(rows < N, lax.rsqrt(deg), 0.0)


def _tc_first(deg_ref, x_ref, w_ref, oa_ref, ob_ref):
    t = _dis(deg_ref) * _DOT(x_ref[...], w_ref[...])
    oa_ref[...] = t[:, 0:64]
    ob_ref[...] = t[:, 64:128]


def _tc_mid(aa_ref, ab_ref, ta_ref, tb_ref, deg_ref, w_ref, b_ref,
            *o_refs):
    dis = _dis(deg_ref)
    sa = aa_ref[0] + aa_ref[1] + ta_ref[...]
    sb = ab_ref[0] + ab_ref[1] + tb_ref[...]
    z = dis * jnp.concatenate([sa, sb], axis=1) + b_ref[...]
    h = jnp.maximum(z, 0.0)
    t = dis * _DOT(h, w_ref[...])
    if len(o_refs) == 2:
        o_refs[0][...] = t[:, 0:64]
        o_refs[1][...] = t[:, 64:128]
    else:
        o_refs[0][...] = t


def _tc_last(agg_ref, t_ref, deg_ref, b_ref, o_ref):
    dis = _dis(deg_ref)
    s = agg_ref[0] + agg_ref[1] + t_ref[...]
    o_ref[...] = dis * s + b_ref[...]


def _row_spec(shape):
    if len(shape) == 3:
        return pl.BlockSpec((shape[0], BM, shape[2]), lambda i: (0, i, 0))
    if shape[0] == NP:
        return pl.BlockSpec((BM, shape[1]), lambda i: (i, 0))
    return pl.BlockSpec(shape, lambda i: (0, 0))


def _tc_call(body, out_shapes, *args):
    return pl.pallas_call(
        body,
        grid=(_GRID,),
        in_specs=[_row_spec(a.shape) for a in args],
        out_specs=[_row_spec(s) for s in out_shapes],
        out_shape=[jax.ShapeDtypeStruct(s, jnp.float32) for s in out_shapes],
    )(*args)


def kernel(x, edge_index, W1, b1, W2, b2, W3, b3):
    ei = edge_index.astype(jnp.int32)
    pad = jnp.full((EPAD - E,), N, jnp.int32)
    src = jnp.concatenate([ei[0], pad]).reshape(NW, NCHUNK, CH)
    dst = jnp.concatenate([ei[1], pad]).reshape(NW, NCHUNK, CH)
    xp = jnp.zeros((NP, x.shape[1]), jnp.float32).at[:N].set(x)

    deg_pl = _sc_pass(16, with_gather=False)(dst)
    agg = _sc_pass(64, with_gather=True)

    t1a, t1b = _tc_call(_tc_first, [(NP, 64)] * 2, deg_pl, xp, W1)
    a1a = agg(t1a, src, dst)
    a1b = agg(t1b, src, dst)

    t2a, t2b = _tc_call(_tc_mid, [(NP, 64)] * 2, a1a, a1b, t1a, t1b,
                        deg_pl, W2, b1.reshape(1, -1))
    a2a = agg(t2a, src, dst)
    a2b = agg(t2b, src, dst)

    (t3,) = _tc_call(_tc_mid, [(NP, 64)], a2a, a2b, t2a, t2b,
                     deg_pl, W3, b2.reshape(1, -1))
    a3 = agg(t3, src, dst)

    (out,) = _tc_call(_tc_last, [(NP, 64)], a3, t3, deg_pl, b3.reshape(1, -1))
    return out[:N]


# speedup vs baseline: 2.8845x; 2.8845x over previous
import functools

import jax
import jax.numpy as jnp
from jax import lax
from jax.experimental import pallas as pl
from jax.experimental.pallas import tpu as pltpu
from jax.experimental.pallas import tpu_sc as plsc

N = 10000
E = 320000
NP = 10112
NC, NS = 2, 16
NW = NC * NS
CH = 128
NB = 5
NCHUNK = 80
NR = NCHUNK // NB
EW = NCHUNK * CH
EPAD = NW * EW
RPT = NP // NS


def _sc_pass(D, with_gather):
    mesh = plsc.VectorSubcoreMesh(core_axis_name="c", subcore_axis_name="s")

    def body(*refs):
        if with_gather:
            (table_hbm, src_hbm, dst_hbm, out_hbm,
             rows_v, srcs_v, dsts_v, acc_sh, gsem, ssem) = refs
        else:
            dst_hbm, out_hbm, rows_v, dsts_v, acc_sh, ssem = refs
        c = lax.axis_index("c")
        s = lax.axis_index("s")
        wid = c * NS + s

        @pl.loop(0, CH)
        def _(i):
            @pl.loop(0, D // 16)
            def _(j):
                rows_v[0, i, pl.ds(j * 16, 16)] = jnp.zeros((16,), jnp.float32)

        r0 = s * RPT

        @pl.loop(0, RPT // CH)
        def _(k):
            pltpu.sync_copy(rows_v.at[0], acc_sh.at[pl.ds(r0 + k * CH, CH)])

        rem = RPT % CH
        if rem:
            pltpu.sync_copy(
                rows_v.at[0, pl.ds(0, rem)],
                acc_sh.at[pl.ds(r0 + (RPT // CH) * CH, rem)],
            )

        if not with_gather:
            @pl.loop(0, CH)
            def _(i):
                @pl.loop(0, D // 16)
                def _(j):
                    rows_v[0, i, pl.ds(j * 16, 16)] = jnp.ones((16,), jnp.float32)

        pltpu.sync_copy(dst_hbm.at[wid], dsts_v)
        if with_gather:
            pltpu.sync_copy(src_hbm.at[wid], srcs_v)

        plsc.subcore_barrier()

        if with_gather:
            for b in range(NB):
                pltpu.async_copy(
                    table_hbm.at[srcs_v.at[b]], rows_v.at[b], gsem.at[b]
                )

            @pl.loop(0, NR)
            def _(r):
                j0 = r * NB
                for b in range(NB):
                    pltpu.make_async_copy(
                        table_hbm.at[srcs_v.at[j0 + b]], rows_v.at[b],
                        gsem.at[b],
                    ).wait()
                    pltpu.async_copy(
                        rows_v.at[b], acc_sh.at[dsts_v.at[j0 + b]],
                        ssem.at[b], add=True,
                    )
                for b in range(NB):
                    pltpu.make_async_copy(
                        rows_v.at[b], acc_sh.at[dsts_v.at[j0 + b]],
                        ssem.at[b],
                    ).wait()

                    @pl.when(r < NR - 1)
                    def _():
                        pltpu.async_copy(
                            table_hbm.at[srcs_v.at[j0 + NB + b]],
                            rows_v.at[b], gsem.at[b],
                        )
        else:
            @pl.loop(0, NR)
            def _(r):
                j0 = r * NB
                for b in range(NB):
                    pltpu.async_copy(
                        rows_v.at[0], acc_sh.at[dsts_v.at[j0 + b]],
                        ssem.at[b], add=True,
                    )
                for b in range(NB):
                    pltpu.make_async_copy(
                        rows_v.at[0], acc_sh.at[dsts_v.at[j0 + b]],
                        ssem.at[b],
                    ).wait()

        plsc.subcore_barrier()

        @pl.loop(0, RPT // CH)
        def _(k):
            pltpu.sync_copy(
                acc_sh.at[pl.ds(r0 + k * CH, CH)],
                out_hbm.at[c, pl.ds(r0 + k * CH, CH)],
            )

        if rem:
            pltpu.sync_copy(
                acc_sh.at[pl.ds(r0 + (RPT // CH) * CH, rem)],
                out_hbm.at[c, pl.ds(r0 + (RPT // CH) * CH, rem)],
            )

    if with_gather:
        scratch = [
            pltpu.VMEM((NB, CH, D), jnp.float32),
            pltpu.VMEM((NCHUNK, CH), jnp.int32),
            pltpu.VMEM((NCHUNK, CH), jnp.int32),
            pltpu.VMEM_SHARED((NP, D), jnp.float32),
            pltpu.SemaphoreType.DMA((NB,)),
            pltpu.SemaphoreType.DMA((NB,)),
        ]
    else:
        scratch = [
            pltpu.VMEM((1, CH, D), jnp.float32),
            pltpu.VMEM((NCHUNK, CH), jnp.int32),
            pltpu.VMEM_SHARED((NP, D), jnp.float32),
            pltpu.SemaphoreType.DMA((NB,)),
        ]
    return pl.kernel(
        body,
        out_type=jax.ShapeDtypeStruct((NC, NP, D), jnp.float32),
        mesh=mesh,
        compiler_params=pltpu.CompilerParams(use_tc_tiling_on_sc=False),
        scratch_types=scratch,
    )


_DOT = functools.partial(
    lax.dot_general,
    dimension_numbers=(((1,), (0,)), ((), ())),
    precision=lax.Precision.HIGHEST,
    preferred_element_type=jnp.float32,
)


BM = 1264
_GRID = NP // BM


def _dis(deg_ref):
    deg = deg_ref[0, :, 0:1] + deg_ref[1, :, 0:1] + 1.0
    r0 = pl.program_id(0) * BM
    rows = r0 + lax.broadcasted_iota(jnp.int32, (BM, 1), 0)
    return jnp.where(rows < N, lax.rsqrt(deg), 0.0)


def _tc_first(deg_ref, x_ref, w_ref, oa_ref, ob_ref):
    t = _dis(deg_ref) * _DOT(x_ref[...], w_ref[...])
    oa_ref[...] = t[:, 0:64]
    ob_ref[...] = t[:, 64:128]


def _tc_mid(aa_ref, ab_ref, ta_ref, tb_ref, deg_ref, w_ref, b_ref,
            *o_refs):
    dis = _dis(deg_ref)
    sa = aa_ref[0] + aa_ref[1] + ta_ref[...]
    sb = ab_ref[0] + ab_ref[1] + tb_ref[...]
    z = dis * jnp.concatenate([sa, sb], axis=1) + b_ref[...]
    h = jnp.maximum(z, 0.0)
    t = dis * _DOT(h, w_ref[...])
    if len(o_refs) == 2:
        o_refs[0][...] = t[:, 0:64]
        o_refs[1][...] = t[:, 64:128]
    else:
        o_refs[0][...] = t


def _tc_last(agg_ref, t_ref, deg_ref, b_ref, o_ref):
    dis = _dis(deg_ref)
    s = agg_ref[0] + agg_ref[1] + t_ref[...]
    o_ref[...] = dis * s + b_ref[...]


def _row_spec(shape):
    if len(shape) == 3:
        return pl.BlockSpec((shape[0], BM, shape[2]), lambda i: (0, i, 0))
    if shape[0] == NP:
        return pl.BlockSpec((BM, shape[1]), lambda i: (i, 0))
    return pl.BlockSpec(shape, lambda i: (0, 0))


def _tc_call(body, out_shapes, *args):
    return pl.pallas_call(
        body,
        grid=(_GRID,),
        in_specs=[_row_spec(a.shape) for a in args],
        out_specs=[_row_spec(s) for s in out_shapes],
        out_shape=[jax.ShapeDtypeStruct(s, jnp.float32) for s in out_shapes],
    )(*args)


def kernel(x, edge_index, W1, b1, W2, b2, W3, b3):
    ei = edge_index.astype(jnp.int32)
    pad = N + jnp.arange(EPAD - E, dtype=jnp.int32) % (NP - N)
    src = jnp.concatenate([ei[0], pad]).reshape(NW, NCHUNK, CH)
    dst = jnp.concatenate([ei[1], pad]).reshape(NW, NCHUNK, CH)
    xp = jnp.zeros((NP, x.shape[1]), jnp.float32).at[:N].set(x)

    deg_pl = _sc_pass(16, with_gather=False)(dst)
    agg = _sc_pass(64, with_gather=True)

    t1a, t1b = _tc_call(_tc_first, [(NP, 64)] * 2, deg_pl, xp, W1)
    a1a = agg(t1a, src, dst)
    a1b = agg(t1b, src, dst)

    t2a, t2b = _tc_call(_tc_mid, [(NP, 64)] * 2, a1a, a1b, t1a, t1b,
                        deg_pl, W2, b1.reshape(1, -1))
    a2a = agg(t2a, src, dst)
    a2b = agg(t2b, src, dst)

    (t3,) = _tc_call(_tc_mid, [(NP, 64)], a2a, a2b, t2a, t2b,
                     deg_pl, W3, b2.reshape(1, -1))
    a3 = agg(t3, src, dst)

    (out,) = _tc_call(_tc_last, [(NP, 64)], a3, t3, deg_pl, b3.reshape(1, -1))
    return out[:N]
